# Initial kernel scaffold; baseline (speedup 1.0000x reference)
import jax
import jax.numpy as jnp
from jax.experimental import pallas as pl


def kernel(x_clim, x_poll, x_soc, edge_index, params):
    raise NotImplementedError("write your pallas kernel here")


# speedup vs baseline: 3.7410x; 3.7410x over previous
import functools

import jax
import jax.numpy as jnp
from jax import lax
from jax.experimental import pallas as pl
from jax.experimental.pallas import tpu as pltpu
from jax.experimental.pallas import tpu_sc as plsc

_N = 50000
_E = 800000
_H = 96
_C = 32
_LANES = 128
_GS = 8
_NTILE = 16
_EPAD = 802816
_ROWS = _EPAD // _LANES
_TROWS = _ROWS // _NTILE
_NPAD = 50176
_Q = _NPAD // _NTILE
_R = 2000
_GRID = _N // _R


def _gelu(x):
    return 0.5 * x * (1.0 + lax.erf(x * 0.7071067811865476))


def _row_spec(cols):
    return pl.BlockSpec((_R, cols), lambda i: (i, 0))


def _full_spec(a):
    return pl.BlockSpec(a.shape, lambda i: (0,) * a.ndim)


def _tc_call(body, ins, n_out, out_cols):
    out_shape = [jax.ShapeDtypeStruct((_N, c), jnp.float32) for c in out_cols]
    in_specs = []
    for a, rows in ins:
        in_specs.append(_row_spec(a.shape[1]) if rows else _full_spec(a))
    return pl.pallas_call(
        body,
        grid=(_GRID,),
        in_specs=in_specs,
        out_specs=[_row_spec(c) for c in out_cols],
        out_shape=out_shape,
        compiler_params=pltpu.CompilerParams(
            dimension_semantics=("parallel",)),
    )(*[a for a, _ in ins])


def _enc_body(xc, xp, xs, ecW1, ecb1, ecW2, ecb2, epW1, epb1, epW2, epb2,
              esW1, esb1, esW2, esb2, preW, preb, o0, o1, o2):
    hc = _gelu(xc[...] @ ecW1[...] + ecb1[...]) @ ecW2[...] + ecb2[...]
    hp = _gelu(xp[...] @ epW1[...] + epb1[...]) @ epW2[...] + epb2[...]
    hs = _gelu(xs[...] @ esW1[...] + esb1[...]) @ esW2[...] + esb2[...]
    h = jnp.concatenate([hc, hp, hs], axis=-1)
    h = _gelu(h @ preW[...] + preb[...])
    o0[...] = h[:, :_C]
    o1[...] = h[:, _C:2 * _C]
    o2[...] = h[:, 2 * _C:]


def _cmb_body(a0, a1, a2, cnt, h0, h1, h2, Wl, bl, Wr, br, o0, o1, o2):
    agg = jnp.concatenate([a0[...], a1[...], a2[...]], axis=-1)
    mean = agg / jnp.maximum(cnt[...], 1.0)
    h = jnp.concatenate([h0[...], h1[...], h2[...]], axis=-1)
    o = _gelu(mean @ Wl[...] + h @ Wr[...] + bl[...] + br[...])
    o0[...] = o[:, :_C]
    o1[...] = o[:, _C:2 * _C]
    o2[...] = o[:, 2 * _C:]


def _fin_body(a0, a1, a2, cnt, h0, h1, h2, Wl, bl, Wr, br, postW, postb,
              dcW1, dcb1, dcW2, dcb2, dpW1, dpb1, dpW2, dpb2,
              dsW1, dsb1, dsW2, dsb2, oz, oxc, oxp, oxs):
    agg = jnp.concatenate([a0[...], a1[...], a2[...]], axis=-1)
    mean = agg / jnp.maximum(cnt[...], 1.0)
    h = jnp.concatenate([h0[...], h1[...], h2[...]], axis=-1)
    g = _gelu(mean @ Wl[...] + h @ Wr[...] + bl[...] + br[...])
    z = g @ postW[...] + postb[...]
    oz[...] = z
    zc = z[:, :_C]
    zp = z[:, _C:2 * _C]
    zs = z[:, 2 * _C:]
    oxc[...] = _gelu(zc @ dcW1[...] + dcb1[...]) @ dcW2[...] + dcb2[...]
    oxp[...] = _gelu(zp @ dpW1[...] + dpb1[...]) @ dpW2[...] + dpb2[...]
    oxs[...] = _gelu(zs @ dsW1[...] + dsb1[...]) @ dsW2[...] + dsb2[...]


def _make_sc_agg(with_cnt):
    mesh = plsc.VectorSubcoreMesh(core_axis_name="c", subcore_axis_name="s",
                                  num_cores=2, num_subcores=_NTILE)
    out_type = [jax.ShapeDtypeStruct((_NPAD, _C), jnp.float32)
                for _ in range(3)]
    scratch = [
        pltpu.VMEM((_GS, _LANES), jnp.int32),
        pltpu.VMEM((_GS, _LANES), jnp.int32),
        pltpu.VMEM((_LANES, _C), jnp.float32),
        pltpu.VMEM_SHARED((_NPAD, _C), jnp.float32),
        pltpu.SemaphoreType.DMA,
    ]
    if with_cnt:
        out_type.append(jax.ShapeDtypeStruct((_NPAD,), jnp.float32))
        scratch.append(pltpu.VMEM((_LANES,), jnp.float32))
        scratch.append(pltpu.VMEM_SHARED((_NPAD,), jnp.float32))

    def body(t0h, t1h, t2h, srch, dsth, z2dh, z1dh, onesh, *rest):
        if with_cnt:
            (o0, o1, o2, ocnt, src_v, dst_v, rows_v, agg_sh, sem,
             ones_v, cnt_sh) = rest
        else:
            o0, o1, o2, src_v, dst_v, rows_v, agg_sh, sem = rest
        c = lax.axis_index("c")
        s = lax.axis_index("s")
        row0 = s * _TROWS
        q0 = s * _Q

        def accum(table_h, out_h):
            pltpu.sync_copy(z2dh, agg_sh.at[pl.ds(q0, _Q)])
            plsc.subcore_barrier()

            def grp(g, carry):
                base = row0 + g * _GS
                pltpu.sync_copy(srch.at[pl.ds(base, _GS)], src_v)
                pltpu.sync_copy(dsth.at[pl.ds(base, _GS)], dst_v)
                for j in range(_GS):
                    pltpu.async_copy(table_h.at[src_v.at[j]], rows_v,
                                     sem).wait()
                    pltpu.sync_copy(rows_v, agg_sh.at[dst_v.at[j]], add=True)
                return carry

            lax.fori_loop(0, _TROWS // _GS, grp, 0)
            plsc.subcore_barrier()
            pltpu.sync_copy(agg_sh.at[pl.ds(q0, _Q)], out_h.at[pl.ds(q0, _Q)])

        def cnt_accum():
            pltpu.sync_copy(z1dh, cnt_sh.at[pl.ds(q0, _Q)])
            pltpu.sync_copy(onesh, ones_v)
            plsc.subcore_barrier()

            def grp(g, carry):
                base = row0 + g * _GS
                pltpu.sync_copy(dsth.at[pl.ds(base, _GS)], dst_v)
                for j in range(_GS):
                    pltpu.sync_copy(ones_v, cnt_sh.at[dst_v.at[j]], add=True)
                return carry

            lax.fori_loop(0, _TROWS // _GS, grp, 0)
            plsc.subcore_barrier()
            pltpu.sync_copy(cnt_sh.at[pl.ds(q0, _Q)],
                            ocnt.at[pl.ds(q0, _Q)])

        @pl.when(c == 0)
        def _():
            accum(t0h, o0)

        @pl.when(c == 1)
        def _():
            accum(t1h, o1)

        @pl.when(c == 0)
        def _():
            accum(t2h, o2)

        if with_cnt:
            @pl.when(c == 1)
            def _():
                cnt_accum()

    return pl.kernel(body, out_type=out_type, mesh=mesh,
                     scratch_types=scratch,
                     compiler_params=pltpu.CompilerParams(
                         use_tc_tiling_on_sc=False))


@functools.cache
def _sc_agg_cached(with_cnt):
    return _make_sc_agg(with_cnt)


def kernel(x_clim, x_poll, x_soc, edge_index, params):
    p = params

    def b(name):
        return p[name].reshape(1, -1)

    h0 = _tc_call(
        _enc_body,
        [(x_clim, True), (x_poll, True), (x_soc, True),
         (p["ec_W1"], False), (b("ec_b1"), False), (p["ec_W2"], False),
         (b("ec_b2"), False),
         (p["ep_W1"], False), (b("ep_b1"), False), (p["ep_W2"], False),
         (b("ep_b2"), False),
         (p["es_W1"], False), (b("es_b1"), False), (p["es_W2"], False),
         (b("es_b2"), False),
         (p["pre_W"], False), (b("pre_b"), False)],
        3, [_C, _C, _C])

    src = edge_index[0]
    dst = edge_index[1]
    pad = _EPAD - _E
    srcR = jnp.concatenate(
        [src, jnp.zeros((pad,), jnp.int32)]).reshape(_ROWS, _LANES)
    dstR = jnp.concatenate(
        [dst, jnp.full((pad,), _N, jnp.int32)]).reshape(_ROWS, _LANES)
    z2d = jnp.zeros((_Q, _C), jnp.float32)
    z1d = jnp.zeros((_Q,), jnp.float32)
    ones = jnp.ones((_LANES,), jnp.float32)

    a0, a1, a2, cnt = _sc_agg_cached(True)(h0[0], h0[1], h0[2], srcR, dstR,
                                           z2d, z1d, ones)
    cnt2 = cnt.reshape(_NPAD, 1)

    h1 = _tc_call(
        _cmb_body,
        [(a0, True), (a1, True), (a2, True), (cnt2, True),
         (h0[0], True), (h0[1], True), (h0[2], True),
         (p["c0_Wl"], False), (b("c0_bl"), False),
         (p["c0_Wr"], False), (b("c0_br"), False)],
        3, [_C, _C, _C])

    b0, b1_, b2 = _sc_agg_cached(False)(h1[0], h1[1], h1[2], srcR, dstR,
                                        z2d, z1d, ones)

    z, xc, xp, xs = _tc_call(
        _fin_body,
        [(b0, True), (b1_, True), (b2, True), (cnt2, True),
         (h1[0], True), (h1[1], True), (h1[2], True),
         (p["c1_Wl"], False), (b("c1_bl"), False),
         (p["c1_Wr"], False), (b("c1_br"), False),
         (p["post_W"], False), (b("post_b"), False),
         (p["dc_W1"], False), (b("dc_b1"), False),
         (p["dc_W2"], False), (b("dc_b2"), False),
         (p["dp_W1"], False), (b("dp_b1"), False),
         (p["dp_W2"], False), (b("dp_b2"), False),
         (p["ds_W1"], False), (b("ds_b1"), False),
         (p["ds_W2"], False), (b("ds_b2"), False)],
        4, [_H, 64, _C, _C])

    return (z, (xc, xp, xs))
